# baseline (speedup 1.0000x reference)
import functools
import jax
import jax.numpy as jnp
from jax import lax
from jax.experimental import pallas as pl
from jax.experimental.pallas import tpu as pltpu, tpu_sc as plsc

_SEQ, _DIM = 8192, 1024
_SCALE = _DIM ** (-0.5)
_NC, _NS = 2, 16
_NW = _NC * _NS
_ROWS_W = _SEQ // _NW
_CH_ROWS = 32
_NCHUNK = _ROWS_W // _CH_ROWS
_NBUF = 3
_CH_E = _CH_ROWS * _DIM

_mesh = plsc.VectorSubcoreMesh(core_axis_name="c", subcore_axis_name="s")


@functools.partial(
    pl.kernel,
    out_type=jax.ShapeDtypeStruct((_SEQ, _DIM), jnp.float32),
    mesh=_mesh,
    scratch_types=[pltpu.VMEM((_NBUF * _CH_ROWS, _DIM), jnp.float32)]
    + [pltpu.SemaphoreType.DMA] * (2 * _NBUF),
)
def _sc_scale(emb_hbm, out_hbm, buf, *sems):
    in_sems = sems[:_NBUF]
    out_sems = sems[_NBUF:]
    wid = lax.axis_index("s") * _NC + lax.axis_index("c")
    base = wid * _ROWS_W
    in_d = [None] * _NCHUNK
    out_d = [None] * _NCHUNK
    for i in range(min(_NBUF, _NCHUNK)):
        in_d[i] = pltpu.async_copy(
            emb_hbm.at[pl.ds(base + i * _CH_ROWS, _CH_ROWS)],
            buf.at[pl.ds(i * _CH_ROWS, _CH_ROWS)],
            in_sems[i],
        )
    for i in range(_NCHUNK):
        b = i % _NBUF
        in_d[i].wait()
        row0 = b * _CH_ROWS

        @plsc.parallel_loop(0, _CH_E, 16, unroll=8)
        def _(j):
            r = row0 + (j >> 10)
            c = pl.multiple_of(j & (_DIM - 1), 16)
            buf[r, pl.ds(c, 16)] = buf[r, pl.ds(c, 16)] * _SCALE

        out_d[i] = pltpu.async_copy(
            buf.at[pl.ds(row0, _CH_ROWS)],
            out_hbm.at[pl.ds(base + i * _CH_ROWS, _CH_ROWS)],
            out_sems[b],
        )
        nxt = i + _NBUF
        if nxt < _NCHUNK:
            out_d[i].wait()
            in_d[nxt] = pltpu.async_copy(
                emb_hbm.at[pl.ds(base + nxt * _CH_ROWS, _CH_ROWS)],
                buf.at[pl.ds(row0, _CH_ROWS)],
                in_sems[b],
            )
    for i in range(max(0, _NCHUNK - _NBUF), _NCHUNK):
        out_d[i].wait()


def kernel(x, emb):
    del x
    return _sc_scale(emb)


# speedup vs baseline: 1.0550x; 1.0550x over previous
import functools
import jax
import jax.numpy as jnp
from jax import lax
from jax.experimental import pallas as pl
from jax.experimental.pallas import tpu as pltpu, tpu_sc as plsc

_SEQ, _DIM = 8192, 1024
_SCALE = _DIM ** (-0.5)
_NC, _NS = 2, 16
_NW = _NC * _NS
_ROWS_W = _SEQ // _NW
_CH_ROWS = 16
_NCHUNK = _ROWS_W // _CH_ROWS
_NBUF = 7
_CH_E = _CH_ROWS * _DIM

_mesh = plsc.VectorSubcoreMesh(core_axis_name="c", subcore_axis_name="s")


@functools.partial(
    pl.kernel,
    out_type=jax.ShapeDtypeStruct((_SEQ, _DIM), jnp.float32),
    mesh=_mesh,
    scratch_types=[pltpu.VMEM((_NBUF * _CH_ROWS, _DIM), jnp.float32)]
    + [pltpu.SemaphoreType.DMA] * (2 * _NBUF),
)
def _sc_scale(emb_hbm, out_hbm, buf, *sems):
    in_sems = sems[:_NBUF]
    out_sems = sems[_NBUF:]
    wid = lax.axis_index("s") * _NC + lax.axis_index("c")
    base = wid * _ROWS_W
    in_d = [None] * _NCHUNK
    out_d = [None] * _NCHUNK
    for i in range(min(_NBUF, _NCHUNK)):
        in_d[i] = pltpu.async_copy(
            emb_hbm.at[pl.ds(base + i * _CH_ROWS, _CH_ROWS)],
            buf.at[pl.ds(i * _CH_ROWS, _CH_ROWS)],
            in_sems[i],
        )
    for i in range(_NCHUNK):
        b = i % _NBUF
        in_d[i].wait()
        row0 = b * _CH_ROWS

        @plsc.parallel_loop(0, _CH_E, 16, unroll=8)
        def _(j):
            r = row0 + (j >> 10)
            c = pl.multiple_of(j & (_DIM - 1), 16)
            buf[r, pl.ds(c, 16)] = buf[r, pl.ds(c, 16)] * _SCALE

        out_d[i] = pltpu.async_copy(
            buf.at[pl.ds(row0, _CH_ROWS)],
            out_hbm.at[pl.ds(base + i * _CH_ROWS, _CH_ROWS)],
            out_sems[b],
        )
        nxt = i + _NBUF
        if nxt < _NCHUNK:
            out_d[i].wait()
            in_d[nxt] = pltpu.async_copy(
                emb_hbm.at[pl.ds(base + nxt * _CH_ROWS, _CH_ROWS)],
                buf.at[pl.ds(row0, _CH_ROWS)],
                in_sems[b],
            )
    for i in range(max(0, _NCHUNK - _NBUF), _NCHUNK):
        out_d[i].wait()


def kernel(x, emb):
    del x
    return _sc_scale(emb)
